# baseline (speedup 1.0000x reference)
import functools

import jax
import jax.numpy as jnp
from jax import lax
from jax.experimental import pallas as pl
from jax.experimental.pallas import tpu as pltpu
from jax.experimental.pallas import tpu_sc as plsc

B = 16384
D = 32
L = 16
NC = 2
NS = 16
NW = NC * NS
BPW = B // NW
NG = BPW // L

_mesh = plsc.VectorSubcoreMesh(core_axis_name="c", subcore_axis_name="s")


@functools.partial(
    pl.kernel,
    out_type=jax.ShapeDtypeStruct((B,), jnp.float32),
    mesh=_mesh,
    compiler_params=pltpu.CompilerParams(
        needs_layout_passes=False, use_tc_tiling_on_sc=False),
    scratch_types=[
        pltpu.VMEM((BPW,), jnp.int32),
        pltpu.VMEM((BPW,), jnp.int32),
        pltpu.VMEM((BPW, D), jnp.float32),
        pltpu.VMEM((BPW, D), jnp.float32),
        pltpu.VMEM((BPW,), jnp.float32),
        pltpu.VMEM((BPW,), jnp.float32),
        pltpu.VMEM((BPW,), jnp.float32),
        pltpu.SemaphoreType.DMA,
    ],
)
def _gmf_sc(v_idx_hbm, h_idx_hbm, virus_hbm, human_hbm, vb_hbm, hb_hbm,
            out_hbm, idx_v, idx_h, rows_v, rows_h, bias_v, bias_h, out_vm,
            sem):
    wid = lax.axis_index("s") * NC + lax.axis_index("c")
    base = pl.multiple_of(wid * BPW, BPW)

    pltpu.sync_copy(v_idx_hbm.at[pl.ds(base, BPW)], idx_v)
    pltpu.sync_copy(h_idx_hbm.at[pl.ds(base, BPW)], idx_h)

    cv = pltpu.async_copy(virus_hbm.at[idx_v], rows_v, sem)
    ch = pltpu.async_copy(human_hbm.at[idx_h], rows_h, sem)
    cbv = pltpu.async_copy(vb_hbm.at[idx_v], bias_v, sem)
    cbh = pltpu.async_copy(hb_hbm.at[idx_h], bias_h, sem)
    cv.wait()
    ch.wait()
    cbv.wait()
    cbh.wait()

    lanes = lax.iota(jnp.int32, L)

    def group_body(g, carry):
        row = g * L + lanes
        start = pl.multiple_of(g * L, L)
        acc0 = bias_v[pl.ds(start, L)]
        acc1 = bias_h[pl.ds(start, L)]
        acc2 = jnp.zeros((L,), jnp.float32)
        acc3 = jnp.zeros((L,), jnp.float32)
        accs = [acc0, acc1, acc2, acc3]
        for d in range(D):
            col = jnp.full((L,), d, jnp.int32)
            xv = plsc.load_gather(rows_v, [row, col])
            xh = plsc.load_gather(rows_h, [row, col])
            accs[d % 4] = accs[d % 4] + xv * xh
        out_vm[pl.ds(start, L)] = (accs[0] + accs[1]) + (accs[2] + accs[3])
        return carry

    lax.fori_loop(0, NG, group_body, 0)

    pltpu.sync_copy(out_vm, out_hbm.at[pl.ds(base, BPW)])


def kernel(v_idxs, h_idxs, virus, human, virus_b, human_b):
    return _gmf_sc(v_idxs, h_idxs, virus, human,
                   virus_b.reshape((-1,)), human_b.reshape((-1,)))
